# Initial kernel scaffold; baseline (speedup 1.0000x reference)
import functools

import jax
import jax.numpy as jnp
from jax import lax
from jax.experimental import pallas as pl
from jax.experimental.pallas import tpu as pltpu
from jax.experimental.pallas import tpu_sc as plsc

NN = 10000
EE = 160000
D_IN = 256
D_H = 512

NC = 2
NS = 16
L = 16
CHUNK = 128

E_PER_S = EE // NS
B = 80
NB = E_PER_S // B
ROWS_PER_S = NN // NS
ZROWS = 125


def _sc_aggregate_body(nchunk, with_deg, *refs):
    h_refs = refs[:nchunk]
    src_hbm, dst_hbm, ew_hbm = refs[nchunk:nchunk + 3]
    agg_out = refs[nchunk + 3]
    i = nchunk + 4
    if with_deg:
        deg_out = refs[i]
        i += 1
    (src_v, dst_v, ew_v, rows_v, ones_v, zbuf, zdeg,
     spmem_agg, spmem_deg, gsem) = refs[i:]

    c = lax.axis_index("c")
    s = lax.axis_index("s")

    zero16 = jnp.zeros((L,), jnp.float32)
    one16 = jnp.ones((L,), jnp.float32)

    def zb_body(r, _):
        for k in range(CHUNK // L):
            zbuf[r, pl.ds(k * L, L)] = zero16
        return 0

    lax.fori_loop(0, ZROWS, zb_body, 0)

    if with_deg:
        def zd_body(r, _):
            zdeg[r, :] = zero16
            return 0

        lax.fori_loop(0, ROWS_PER_S, zd_body, 0)

        def on_body(r, _):
            ones_v[r, :] = one16
            return 0

        lax.fori_loop(0, B, on_body, 0)

    pltpu.sync_copy(src_hbm.at[s], src_v)
    pltpu.sync_copy(dst_hbm.at[s], dst_v)
    pltpu.sync_copy(ew_hbm.at[s], ew_v)

    for q in range(nchunk):
        owner = q % NC
        do_deg = with_deg and q == 0

        @pl.when(c == owner)
        def _pass(q=q, do_deg=do_deg):
            hc = h_refs[q]

            for t in range(ROWS_PER_S // ZROWS):
                pltpu.sync_copy(
                    zbuf, spmem_agg.at[pl.ds(s * ROWS_PER_S + t * ZROWS, ZROWS)])
            if do_deg:
                pltpu.sync_copy(zdeg, spmem_deg.at[pl.ds(s * ROWS_PER_S, ROWS_PER_S)])
            plsc.subcore_barrier()

            def batch_body(j, _):
                pltpu.async_copy(hc.at[src_v.at[j]], rows_v, gsem).wait()

                def mul_body(i, _):
                    jj = jnp.full((L,), j, jnp.int32)
                    ii = jnp.full((L,), i, jnp.int32)
                    w = plsc.load_gather(ew_v, [jj, ii])
                    for k in range(CHUNK // L):
                        sl = pl.ds(k * L, L)
                        rows_v[i, sl] = rows_v[i, sl] * w
                    return 0

                lax.fori_loop(0, B, mul_body, 0)

                pltpu.sync_copy(rows_v, spmem_agg.at[dst_v.at[j]], add=True)
                if do_deg:
                    pltpu.sync_copy(ones_v, spmem_deg.at[dst_v.at[j]], add=True)
                return 0

            lax.fori_loop(0, NB, batch_body, 0)
            plsc.subcore_barrier()

            pltpu.sync_copy(
                spmem_agg.at[pl.ds(s * ROWS_PER_S, ROWS_PER_S)],
                agg_out.at[q, pl.ds(s * ROWS_PER_S, ROWS_PER_S)])
            if do_deg:
                pltpu.sync_copy(
                    spmem_deg.at[pl.ds(s * ROWS_PER_S, ROWS_PER_S)],
                    deg_out.at[pl.ds(s * ROWS_PER_S, ROWS_PER_S)])


def _make_sc_aggregate(nchunk, with_deg):
    out_type = [jax.ShapeDtypeStruct((nchunk, NN, CHUNK), jnp.float32)]
    if with_deg:
        out_type.append(jax.ShapeDtypeStruct((NN, L), jnp.float32))
    scratch = [
        pltpu.VMEM((NB, B), jnp.int32),
        pltpu.VMEM((NB, B), jnp.int32),
        pltpu.VMEM((NB, B), jnp.float32),
        pltpu.VMEM((B, CHUNK), jnp.float32),
        pltpu.VMEM((B, L), jnp.float32),
        pltpu.VMEM((ZROWS, CHUNK), jnp.float32),
        pltpu.VMEM((ROWS_PER_S, L), jnp.float32),
        pltpu.VMEM_SHARED((NN, CHUNK), jnp.float32),
        pltpu.VMEM_SHARED((NN, L), jnp.float32),
        pltpu.SemaphoreType.DMA,
    ]
    mesh = plsc.VectorSubcoreMesh(
        core_axis_name="c", subcore_axis_name="s",
        num_cores=NC, num_subcores=NS)
    return pl.kernel(
        functools.partial(_sc_aggregate_body, nchunk, with_deg),
        out_type=tuple(out_type) if with_deg else out_type[0],
        mesh=mesh,
        scratch_types=scratch,
    )


def _tc_layer_body(nchunk, nout, refs_in):
    agg = refs_in[0]
    h_refs = refs_in[1:1 + nchunk]
    deg = refs_in[1 + nchunk]
    W = refs_in[2 + nchunk]
    outs = refs_in[3 + nchunk:]
    a = jnp.concatenate([agg[q] for q in range(nchunk)], axis=1)
    x = jnp.concatenate([h_refs[q][...] for q in range(nchunk)], axis=1)
    scale = 1.0 / (deg[:, 0:1] + 1.0)
    y = jnp.dot((a + x) * scale, W[...], preferred_element_type=jnp.float32)
    y = jnp.maximum(y, 0.0)
    if nout == 1:
        outs[0][...] = y
    else:
        w = y.shape[1] // nout
        for q in range(nout):
            outs[q][...] = y[:, q * w:(q + 1) * w]


def _make_tc_layer(nchunk, d_in, nout, bn=1000):
    grid = (NN // bn,)
    in_specs = [
        pl.BlockSpec((nchunk, bn, CHUNK), lambda i: (0, i, 0)),
    ]
    in_specs += [pl.BlockSpec((bn, CHUNK), lambda i: (i, 0))
                 for _ in range(nchunk)]
    in_specs += [
        pl.BlockSpec((bn, L), lambda i: (i, 0)),
        pl.BlockSpec((d_in, D_H), lambda i: (0, 0)),
    ]
    if nout == 1:
        out_shape = jax.ShapeDtypeStruct((NN, D_H), jnp.float32)
        out_specs = pl.BlockSpec((bn, D_H), lambda i: (i, 0))
    else:
        w = D_H // nout
        out_shape = [jax.ShapeDtypeStruct((NN, w), jnp.float32)
                     for _ in range(nout)]
        out_specs = [pl.BlockSpec((bn, w), lambda i: (i, 0))
                     for _ in range(nout)]

    def body(*refs):
        _tc_layer_body(nchunk, nout, refs)

    return pl.pallas_call(
        body, grid=grid, in_specs=in_specs,
        out_specs=out_specs, out_shape=out_shape)


@jax.jit
def kernel(in_feat, edge_index, edge_weights, W1, W2):
    src = edge_index[0].reshape(NS, NB, B)
    dst = edge_index[1].reshape(NS, NB, B)
    ew = edge_weights.reshape(NS, NB, B)

    x0 = in_feat[:, :CHUNK]
    x1 = in_feat[:, CHUNK:]

    sc1 = _make_sc_aggregate(2, True)
    agg1, deg = sc1(x0, x1, src, dst, ew)

    tc1 = _make_tc_layer(2, D_IN, 4)
    h1c = tc1(agg1, x0, x1, deg, W1)

    sc2 = _make_sc_aggregate(4, False)
    agg2 = sc2(h1c[0], h1c[1], h1c[2], h1c[3], src, dst, ew)

    tc2 = _make_tc_layer(4, D_H, 1)
    out = tc2(agg2, h1c[0], h1c[1], h1c[2], h1c[3], deg, W2)
    return out


# speedup vs baseline: 2.1586x; 2.1586x over previous
import functools

import jax
import jax.numpy as jnp
from jax import lax
from jax.experimental import pallas as pl
from jax.experimental.pallas import tpu as pltpu
from jax.experimental.pallas import tpu_sc as plsc

NN = 10000
EE = 160000
D_IN = 256
D_H = 512

NC = 2
NS = 16
L = 16
CHUNK = 128

E_PER_S = EE // NS
B = 80
NB = E_PER_S // B
NB0 = 63
RPS = 624
EXTRA = NN - NS * RPS
ZROWS = 16


def _sc_aggregate_body(nchunk, with_deg, *refs):
    h_refs = refs[:nchunk]
    src_hbm, dst_hbm, ew_hbm = refs[nchunk:nchunk + 3]
    agg_out = refs[nchunk + 3]
    (src_v, dst_v, ew_v, grow_v, srow_v, zbuf, spmem_acc, gsem) = refs[nchunk + 4:]

    c = lax.axis_index("c")
    s = lax.axis_index("s")

    zero16 = jnp.zeros((L,), jnp.float32)
    one16 = jnp.ones((L,), jnp.float32)

    def zb_body(r, _):
        for k in range(CHUNK // L):
            zbuf[r, pl.ds(k * L, L)] = zero16
        return 0

    lax.fori_loop(0, ZROWS, zb_body, 0)

    dnums = lax.GatherDimensionNumbers(
        offset_dims=(), collapsed_slice_dims=(0,), start_index_map=(0,))

    def zero_spmem():
        def zero_body(t, _):
            pltpu.sync_copy(
                zbuf, spmem_acc.at[pl.ds(s * RPS + t * ZROWS, ZROWS)])
            return 0

        lax.fori_loop(0, RPS // ZROWS, zero_body, 0)

        @pl.when(s == NS - 1)
        def _zero_tail():
            pltpu.sync_copy(zbuf, spmem_acc.at[pl.ds(NS * RPS, EXTRA)])

    def copy_out(slot):
        def copy_body(t, _):
            r0 = s * RPS + t * ZROWS
            pltpu.sync_copy(spmem_acc.at[pl.ds(r0, ZROWS)],
                            agg_out.at[slot, pl.ds(r0, ZROWS)])
            return 0

        lax.fori_loop(0, RPS // ZROWS, copy_body, 0)

        @pl.when(s == NS - 1)
        def _copy_tail():
            pltpu.sync_copy(spmem_acc.at[pl.ds(NS * RPS, EXTRA)],
                            agg_out.at[slot, pl.ds(NS * RPS, EXTRA)])

    for q in range(nchunk):
        owner = q % NC

        @pl.when(c == owner)
        def _pass(q=q):
            hc = h_refs[q]
            zero_spmem()
            plsc.subcore_barrier()

            def batch_body(j, _):
                base = s * E_PER_S + j * B
                pltpu.sync_copy(src_hbm.at[pl.ds(base, B)], src_v)
                pltpu.sync_copy(dst_hbm.at[pl.ds(base, B)], dst_v)
                pltpu.sync_copy(ew_hbm.at[pl.ds(base, B)], ew_v)

                pltpu.async_copy(hc.at[src_v], grow_v, gsem).wait()

                def mul_body(g, _):
                    w16 = ew_v[pl.ds(g * L, L)]
                    for i in range(L):
                        idx = jnp.full((L, 1), i, jnp.int32)
                        w = lax.gather(
                            w16, idx, dnums, (1,),
                            mode=lax.GatherScatterMode.PROMISE_IN_BOUNDS)
                        r = g * L + i
                        for k in range(CHUNK // L):
                            sl = pl.ds(k * L, L)
                            srow_v[r, sl] = grow_v[r, sl] * w
                    return 0

                lax.fori_loop(0, B // L, mul_body, 0)

                pltpu.sync_copy(srow_v, spmem_acc.at[dst_v], add=True)
                return 0

            lax.fori_loop(0, NB, batch_body, 0)
            plsc.subcore_barrier()
            copy_out(q)

    if with_deg:
        def ones_body(r, _):
            for k in range(CHUNK // L):
                srow_v[r, pl.ds(k * L, L)] = one16
            return 0

        lax.fori_loop(0, B, ones_body, 0)
        zero_spmem()
        plsc.subcore_barrier()

        jlo = jnp.where(c == 0, 0, NB0)
        jhi = jnp.where(c == 0, NB0, NB)

        def deg_body(j, _):
            base = s * E_PER_S + j * B
            pltpu.sync_copy(dst_hbm.at[pl.ds(base, B)], dst_v)
            pltpu.sync_copy(srow_v, spmem_acc.at[dst_v], add=True)
            return 0

        lax.fori_loop(jlo, jhi, deg_body, 0)
        plsc.subcore_barrier()
        copy_out(nchunk + c)


def _make_sc_aggregate(nchunk, with_deg):
    nout = nchunk + (NC if with_deg else 0)
    scratch = [
        pltpu.VMEM((B,), jnp.int32),
        pltpu.VMEM((B,), jnp.int32),
        pltpu.VMEM((B,), jnp.float32),
        pltpu.VMEM((B, CHUNK), jnp.float32),
        pltpu.VMEM((B, CHUNK), jnp.float32),
        pltpu.VMEM((ZROWS, CHUNK), jnp.float32),
        pltpu.VMEM_SHARED((NN, CHUNK), jnp.float32),
        pltpu.SemaphoreType.DMA,
    ]
    mesh = plsc.VectorSubcoreMesh(
        core_axis_name="c", subcore_axis_name="s",
        num_cores=NC, num_subcores=NS)
    return pl.kernel(
        functools.partial(_sc_aggregate_body, nchunk, with_deg),
        out_type=jax.ShapeDtypeStruct((nout, NN, CHUNK), jnp.float32),
        mesh=mesh,
        scratch_types=scratch,
    )


def _tc_layer_body(nchunk, nout, refs_in):
    agg = refs_in[0]
    degp = refs_in[1]
    h_refs = refs_in[2:2 + nchunk]
    W = refs_in[2 + nchunk]
    outs = refs_in[3 + nchunk:]
    a = jnp.concatenate([agg[q] for q in range(nchunk)], axis=1)
    x = jnp.concatenate([h_refs[q][...] for q in range(nchunk)], axis=1)
    deg = degp[0, :, 0:1] + degp[1, :, 0:1]
    scale = 1.0 / (deg + 1.0)
    y = jnp.dot((a + x) * scale, W[...], preferred_element_type=jnp.float32)
    y = jnp.maximum(y, 0.0)
    if nout == 1:
        outs[0][...] = y
    else:
        w = y.shape[1] // nout
        for q in range(nout):
            outs[q][...] = y[:, q * w:(q + 1) * w]


def _make_tc_layer(nchunk, nout, bn=1000):
    grid = (NN // bn,)
    in_specs = [
        pl.BlockSpec((nchunk, bn, CHUNK), lambda i: (0, i, 0)),
        pl.BlockSpec((NC, bn, CHUNK), lambda i: (0, i, 0)),
    ]
    in_specs += [pl.BlockSpec((bn, CHUNK), lambda i: (i, 0))
                 for _ in range(nchunk)]
    in_specs += [
        pl.BlockSpec((nchunk * CHUNK, D_H), lambda i: (0, 0)),
    ]
    if nout == 1:
        out_shape = jax.ShapeDtypeStruct((NN, D_H), jnp.float32)
        out_specs = pl.BlockSpec((bn, D_H), lambda i: (i, 0))
    else:
        w = D_H // nout
        out_shape = [jax.ShapeDtypeStruct((NN, w), jnp.float32)
                     for _ in range(nout)]
        out_specs = [pl.BlockSpec((bn, w), lambda i: (i, 0))
                     for _ in range(nout)]

    def body(*refs):
        _tc_layer_body(nchunk, nout, refs)

    return pl.pallas_call(
        body, grid=grid, in_specs=in_specs,
        out_specs=out_specs, out_shape=out_shape)


@jax.jit
def kernel(in_feat, edge_index, edge_weights, W1, W2):
    src = edge_index[0]
    dst = edge_index[1]
    ew = edge_weights

    x0 = in_feat[:, :CHUNK]
    x1 = in_feat[:, CHUNK:]

    sc1 = _make_sc_aggregate(2, True)
    out1 = sc1(x0, x1, src, dst, ew)
    agg1 = out1[:2]
    degp = out1[2:]

    tc1 = _make_tc_layer(2, 4)
    h1c = tc1(agg1, degp, x0, x1, W1)

    sc2 = _make_sc_aggregate(4, False)
    agg2 = sc2(h1c[0], h1c[1], h1c[2], h1c[3], src, dst, ew)

    tc2 = _make_tc_layer(4, 1)
    out = tc2(agg2, degp, h1c[0], h1c[1], h1c[2], h1c[3], W2)
    return out


# speedup vs baseline: 3.6712x; 1.7007x over previous
import functools

import jax
import jax.numpy as jnp
from jax import lax
from jax.experimental import pallas as pl
from jax.experimental.pallas import tpu as pltpu
from jax.experimental.pallas import tpu_sc as plsc

NN = 10000
EE = 160000
D_IN = 256
D_H = 512

NC = 2
NS = 16
L = 16
CHUNK = 128

E_PER_S = EE // NS
B = 80
NB = E_PER_S // B
NB0 = 63
RPS = 624
EXTRA = NN - NS * RPS
ZROWS = 16


def _sc_aggregate_body(nchunk, with_deg, *refs):
    h_refs = refs[:nchunk]
    src_hbm, dst_hbm, ew_hbm = refs[nchunk:nchunk + 3]
    agg_out = refs[nchunk + 3]
    (sv0, dv0, wv0, sv1, dv1, wv1, grow0, grow1, srow_v, zbuf,
     spmem_acc, g0, g1, i0, i1) = refs[nchunk + 4:]
    idx_slots = ((sv0, dv0, wv0, i0), (sv1, dv1, wv1, i1))

    c = lax.axis_index("c")
    s = lax.axis_index("s")

    zero16 = jnp.zeros((L,), jnp.float32)
    one16 = jnp.ones((L,), jnp.float32)

    def zb_body(r, _):
        for k in range(CHUNK // L):
            zbuf[r, pl.ds(k * L, L)] = zero16
        return 0

    lax.fori_loop(0, ZROWS, zb_body, 0)

    dnums = lax.GatherDimensionNumbers(
        offset_dims=(), collapsed_slice_dims=(0,), start_index_map=(0,))

    def zero_spmem():
        def zero_body(t, _):
            pltpu.sync_copy(
                zbuf, spmem_acc.at[pl.ds(s * RPS + t * ZROWS, ZROWS)])
            return 0

        lax.fori_loop(0, RPS // ZROWS, zero_body, 0)

        @pl.when(s == NS - 1)
        def _zero_tail():
            pltpu.sync_copy(zbuf, spmem_acc.at[pl.ds(NS * RPS, EXTRA)])

    def copy_out(slot):
        def copy_body(t, _):
            r0 = s * RPS + t * ZROWS
            pltpu.sync_copy(spmem_acc.at[pl.ds(r0, ZROWS)],
                            agg_out.at[slot, pl.ds(r0, ZROWS)])
            return 0

        lax.fori_loop(0, RPS // ZROWS, copy_body, 0)

        @pl.when(s == NS - 1)
        def _copy_tail():
            pltpu.sync_copy(spmem_acc.at[pl.ds(NS * RPS, EXTRA)],
                            agg_out.at[slot, pl.ds(NS * RPS, EXTRA)])

    def start_idx(j, slot):
        sv, dv, wv, isem = idx_slots[slot]
        base = jnp.minimum(s * E_PER_S + j * B, EE - B)
        pltpu.async_copy(src_hbm.at[pl.ds(base, B)], sv, isem)
        pltpu.async_copy(dst_hbm.at[pl.ds(base, B)], dv, isem)
        pltpu.async_copy(ew_hbm.at[pl.ds(base, B)], wv, isem)

    def wait_idx(slot):
        sv, dv, wv, isem = idx_slots[slot]
        pltpu.make_async_copy(src_hbm.at[pl.ds(0, B)], sv, isem).wait()
        pltpu.make_async_copy(dst_hbm.at[pl.ds(0, B)], dv, isem).wait()
        pltpu.make_async_copy(ew_hbm.at[pl.ds(0, B)], wv, isem).wait()

    def mul_scatter(hc, slot):
        sv, dv, wv, _ = idx_slots[slot]
        grow = grow0 if slot == 0 else grow1

        def mul_body(g, _):
            w16 = wv[pl.ds(g * L, L)]
            for i in range(L):
                idx = jnp.full((L, 1), i, jnp.int32)
                w = lax.gather(
                    w16, idx, dnums, (1,),
                    mode=lax.GatherScatterMode.PROMISE_IN_BOUNDS)
                r = g * L + i
                for k in range(CHUNK // L):
                    sl = pl.ds(k * L, L)
                    srow_v[r, sl] = grow[r, sl] * w
            return 0

        lax.fori_loop(0, B // L, mul_body, 0)
        pltpu.sync_copy(srow_v, spmem_acc.at[dv], add=True)

    for q in range(nchunk):
        owner = q % NC

        @pl.when(c == owner)
        def _pass(q=q):
            hc = h_refs[q]
            zero_spmem()
            plsc.subcore_barrier()

            start_idx(0, 0)
            wait_idx(0)
            pltpu.async_copy(hc.at[sv0], grow0, g0)
            start_idx(1, 1)

            def pair_body(u, _):
                a = 2 * u
                wait_idx(1)
                pltpu.async_copy(hc.at[sv1], grow1, g1)
                pltpu.make_async_copy(hc.at[sv0], grow0, g0).wait()
                mul_scatter(hc, 0)
                start_idx(a + 2, 0)
                pltpu.make_async_copy(hc.at[sv1], grow1, g1).wait()
                mul_scatter(hc, 1)
                start_idx(a + 3, 1)
                wait_idx(0)
                pltpu.async_copy(hc.at[sv0], grow0, g0)
                return 0

            lax.fori_loop(0, (NB - 1) // 2, pair_body, 0)

            wait_idx(1)
            pltpu.make_async_copy(hc.at[sv0], grow0, g0).wait()
            mul_scatter(hc, 0)

            plsc.subcore_barrier()
            copy_out(q)

    if with_deg:
        def ones_body(r, _):
            for k in range(CHUNK // L):
                srow_v[r, pl.ds(k * L, L)] = one16
            return 0

        lax.fori_loop(0, B, ones_body, 0)
        zero_spmem()
        plsc.subcore_barrier()

        jlo = jnp.where(c == 0, 0, NB0)
        jhi = jnp.where(c == 0, NB0, NB)

        def deg_body(j, _):
            base = s * E_PER_S + j * B
            pltpu.sync_copy(dst_hbm.at[pl.ds(base, B)], dv0)
            pltpu.sync_copy(srow_v, spmem_acc.at[dv0], add=True)
            return 0

        lax.fori_loop(jlo, jhi, deg_body, 0)
        plsc.subcore_barrier()
        copy_out(nchunk + c)


def _make_sc_aggregate(nchunk, with_deg):
    nout = nchunk + (NC if with_deg else 0)
    scratch = [
        pltpu.VMEM((B,), jnp.int32),
        pltpu.VMEM((B,), jnp.int32),
        pltpu.VMEM((B,), jnp.float32),
        pltpu.VMEM((B,), jnp.int32),
        pltpu.VMEM((B,), jnp.int32),
        pltpu.VMEM((B,), jnp.float32),
        pltpu.VMEM((B, CHUNK), jnp.float32),
        pltpu.VMEM((B, CHUNK), jnp.float32),
        pltpu.VMEM((B, CHUNK), jnp.float32),
        pltpu.VMEM((ZROWS, CHUNK), jnp.float32),
        pltpu.VMEM_SHARED((NN, CHUNK), jnp.float32),
        pltpu.SemaphoreType.DMA,
        pltpu.SemaphoreType.DMA,
        pltpu.SemaphoreType.DMA,
        pltpu.SemaphoreType.DMA,
    ]
    mesh = plsc.VectorSubcoreMesh(
        core_axis_name="c", subcore_axis_name="s",
        num_cores=NC, num_subcores=NS)
    return pl.kernel(
        functools.partial(_sc_aggregate_body, nchunk, with_deg),
        out_type=jax.ShapeDtypeStruct((nout, NN, CHUNK), jnp.float32),
        mesh=mesh,
        scratch_types=scratch,
    )


def _tc_layer_body(nchunk, nout, refs_in):
    agg = refs_in[0]
    degp = refs_in[1]
    h_refs = refs_in[2:2 + nchunk]
    W = refs_in[2 + nchunk]
    outs = refs_in[3 + nchunk:]
    a = jnp.concatenate([agg[q] for q in range(nchunk)], axis=1)
    x = jnp.concatenate([h_refs[q][...] for q in range(nchunk)], axis=1)
    deg = degp[0, :, 0:1] + degp[1, :, 0:1]
    scale = 1.0 / (deg + 1.0)
    y = jnp.dot((a + x) * scale, W[...], preferred_element_type=jnp.float32)
    y = jnp.maximum(y, 0.0)
    if nout == 1:
        outs[0][...] = y
    else:
        w = y.shape[1] // nout
        for q in range(nout):
            outs[q][...] = y[:, q * w:(q + 1) * w]


def _make_tc_layer(nchunk, nout, bn=1000):
    grid = (NN // bn,)
    in_specs = [
        pl.BlockSpec((nchunk, bn, CHUNK), lambda i: (0, i, 0)),
        pl.BlockSpec((NC, bn, CHUNK), lambda i: (0, i, 0)),
    ]
    in_specs += [pl.BlockSpec((bn, CHUNK), lambda i: (i, 0))
                 for _ in range(nchunk)]
    in_specs += [
        pl.BlockSpec((nchunk * CHUNK, D_H), lambda i: (0, 0)),
    ]
    if nout == 1:
        out_shape = jax.ShapeDtypeStruct((NN, D_H), jnp.float32)
        out_specs = pl.BlockSpec((bn, D_H), lambda i: (i, 0))
    else:
        w = D_H // nout
        out_shape = [jax.ShapeDtypeStruct((NN, w), jnp.float32)
                     for _ in range(nout)]
        out_specs = [pl.BlockSpec((bn, w), lambda i: (i, 0))
                     for _ in range(nout)]

    def body(*refs):
        _tc_layer_body(nchunk, nout, refs)

    return pl.pallas_call(
        body, grid=grid, in_specs=in_specs,
        out_specs=out_specs, out_shape=out_shape)


@jax.jit
def kernel(in_feat, edge_index, edge_weights, W1, W2):
    src = edge_index[0]
    dst = edge_index[1]
    ew = edge_weights

    x0 = in_feat[:, :CHUNK]
    x1 = in_feat[:, CHUNK:]

    sc1 = _make_sc_aggregate(2, True)
    out1 = sc1(x0, x1, src, dst, ew)
    agg1 = out1[:2]
    degp = out1[2:]

    tc1 = _make_tc_layer(2, 4)
    h1c = tc1(agg1, degp, x0, x1, W1)

    sc2 = _make_sc_aggregate(4, False)
    agg2 = sc2(h1c[0], h1c[1], h1c[2], h1c[3], src, dst, ew)

    tc2 = _make_tc_layer(4, 1)
    out = tc2(agg2, degp, h1c[0], h1c[1], h1c[2], h1c[3], W2)
    return out


# speedup vs baseline: 4.1438x; 1.1287x over previous
import functools

import jax
import jax.numpy as jnp
from jax import lax
from jax.experimental import pallas as pl
from jax.experimental.pallas import tpu as pltpu
from jax.experimental.pallas import tpu_sc as plsc

NN = 10000
EE = 160000
D_IN = 256
D_H = 512

NC = 2
NS = 16
L = 16
CHUNK = 128

E_PER_S = EE // NS
B = 80
NB = E_PER_S // B
NB0 = 63
RPS = 624
EXTRA = NN - NS * RPS
ZROWS = 16


def _sc_aggregate_body(nchunk, with_deg, *refs):
    h_refs = refs[:nchunk]
    src_hbm, dst_hbm, ew_hbm = refs[nchunk:nchunk + 3]
    agg_out = refs[nchunk + 3]
    (sv0, dv0, wv0, sv1, dv1, wv1, grow0, grow1, srow_v, zbuf,
     spmem_acc, g0, g1, i0, i1) = refs[nchunk + 4:]
    idx_slots = ((sv0, dv0, wv0, i0), (sv1, dv1, wv1, i1))

    c = lax.axis_index("c")
    s = lax.axis_index("s")

    zero16 = jnp.zeros((L,), jnp.float32)
    one16 = jnp.ones((L,), jnp.float32)

    def zb_body(r, _):
        for k in range(CHUNK // L):
            zbuf[r, pl.ds(k * L, L)] = zero16
        return 0

    lax.fori_loop(0, ZROWS, zb_body, 0)

    dnums = lax.GatherDimensionNumbers(
        offset_dims=(), collapsed_slice_dims=(0,), start_index_map=(0,))

    def zero_spmem():
        def zero_body(t, _):
            pltpu.sync_copy(
                zbuf, spmem_acc.at[pl.ds(s * RPS + t * ZROWS, ZROWS)])
            return 0

        lax.fori_loop(0, RPS // ZROWS, zero_body, 0)

        @pl.when(s == NS - 1)
        def _zero_tail():
            pltpu.sync_copy(zbuf, spmem_acc.at[pl.ds(NS * RPS, EXTRA)])

    def copy_out(slot):
        def copy_body(t, _):
            r0 = s * RPS + t * ZROWS
            pltpu.sync_copy(spmem_acc.at[pl.ds(r0, ZROWS)],
                            agg_out.at[slot, pl.ds(r0, ZROWS)])
            return 0

        lax.fori_loop(0, RPS // ZROWS, copy_body, 0)

        @pl.when(s == NS - 1)
        def _copy_tail():
            pltpu.sync_copy(spmem_acc.at[pl.ds(NS * RPS, EXTRA)],
                            agg_out.at[slot, pl.ds(NS * RPS, EXTRA)])

    def start_idx(j, slot):
        sv, dv, wv, isem = idx_slots[slot]
        base = jnp.minimum(s * E_PER_S + j * B, EE - B)
        pltpu.async_copy(src_hbm.at[pl.ds(base, B)], sv, isem)
        pltpu.async_copy(dst_hbm.at[pl.ds(base, B)], dv, isem)
        pltpu.async_copy(ew_hbm.at[pl.ds(base, B)], wv, isem)

    def wait_idx(slot):
        sv, dv, wv, isem = idx_slots[slot]
        pltpu.make_async_copy(src_hbm.at[pl.ds(0, B)], sv, isem).wait()
        pltpu.make_async_copy(dst_hbm.at[pl.ds(0, B)], dv, isem).wait()
        pltpu.make_async_copy(ew_hbm.at[pl.ds(0, B)], wv, isem).wait()

    def mul_scatter(hc, slot):
        sv, dv, wv, _ = idx_slots[slot]
        grow = grow0 if slot == 0 else grow1

        @plsc.parallel_loop(0, B // L, unroll=2)
        def _mul(g):
            w16 = wv[pl.ds(g * L, L)]
            for i in range(L):
                idx = jnp.full((L, 1), i, jnp.int32)
                w = lax.gather(
                    w16, idx, dnums, (1,),
                    mode=lax.GatherScatterMode.PROMISE_IN_BOUNDS)
                r = g * L + i
                for k in range(CHUNK // L):
                    sl = pl.ds(k * L, L)
                    srow_v[r, sl] = grow[r, sl] * w

        pltpu.sync_copy(srow_v, spmem_acc.at[dv], add=True)

    for q in range(nchunk):
        owner = q % NC

        @pl.when(c == owner)
        def _pass(q=q):
            hc = h_refs[q]
            zero_spmem()
            plsc.subcore_barrier()

            start_idx(0, 0)
            wait_idx(0)
            pltpu.async_copy(hc.at[sv0], grow0, g0)
            start_idx(1, 1)

            def pair_body(u, _):
                a = 2 * u
                wait_idx(1)
                pltpu.async_copy(hc.at[sv1], grow1, g1)
                pltpu.make_async_copy(hc.at[sv0], grow0, g0).wait()
                mul_scatter(hc, 0)
                start_idx(a + 2, 0)
                pltpu.make_async_copy(hc.at[sv1], grow1, g1).wait()
                mul_scatter(hc, 1)
                start_idx(a + 3, 1)
                wait_idx(0)
                pltpu.async_copy(hc.at[sv0], grow0, g0)
                return 0

            lax.fori_loop(0, (NB - 1) // 2, pair_body, 0)

            wait_idx(1)
            pltpu.make_async_copy(hc.at[sv0], grow0, g0).wait()
            mul_scatter(hc, 0)

            plsc.subcore_barrier()
            copy_out(q)

    if with_deg:
        def ones_body(r, _):
            for k in range(CHUNK // L):
                srow_v[r, pl.ds(k * L, L)] = one16
            return 0

        lax.fori_loop(0, B, ones_body, 0)
        zero_spmem()
        plsc.subcore_barrier()

        jlo = jnp.where(c == 0, 0, NB0)
        jhi = jnp.where(c == 0, NB0, NB)

        def deg_body(j, _):
            base = s * E_PER_S + j * B
            pltpu.sync_copy(dst_hbm.at[pl.ds(base, B)], dv0)
            pltpu.sync_copy(srow_v, spmem_acc.at[dv0], add=True)
            return 0

        lax.fori_loop(jlo, jhi, deg_body, 0)
        plsc.subcore_barrier()
        copy_out(nchunk + c)


def _make_sc_aggregate(nchunk, with_deg):
    nout = nchunk + (NC if with_deg else 0)
    scratch = [
        pltpu.VMEM((B,), jnp.int32),
        pltpu.VMEM((B,), jnp.int32),
        pltpu.VMEM((B,), jnp.float32),
        pltpu.VMEM((B,), jnp.int32),
        pltpu.VMEM((B,), jnp.int32),
        pltpu.VMEM((B,), jnp.float32),
        pltpu.VMEM((B, CHUNK), jnp.float32),
        pltpu.VMEM((B, CHUNK), jnp.float32),
        pltpu.VMEM((B, CHUNK), jnp.float32),
        pltpu.VMEM((ZROWS, CHUNK), jnp.float32),
        pltpu.VMEM_SHARED((NN, CHUNK), jnp.float32),
        pltpu.SemaphoreType.DMA,
        pltpu.SemaphoreType.DMA,
        pltpu.SemaphoreType.DMA,
        pltpu.SemaphoreType.DMA,
    ]
    mesh = plsc.VectorSubcoreMesh(
        core_axis_name="c", subcore_axis_name="s",
        num_cores=NC, num_subcores=NS)
    return pl.kernel(
        functools.partial(_sc_aggregate_body, nchunk, with_deg),
        out_type=jax.ShapeDtypeStruct((nout, NN, CHUNK), jnp.float32),
        mesh=mesh,
        scratch_types=scratch,
    )


def _tc_layer_body(nchunk, nout, refs_in):
    agg = refs_in[0]
    degp = refs_in[1]
    h_refs = refs_in[2:2 + nchunk]
    W = refs_in[2 + nchunk]
    outs = refs_in[3 + nchunk:]
    a = jnp.concatenate([agg[q] for q in range(nchunk)], axis=1)
    x = jnp.concatenate([h_refs[q][...] for q in range(nchunk)], axis=1)
    deg = degp[0, :, 0:1] + degp[1, :, 0:1]
    scale = 1.0 / (deg + 1.0)
    y = jnp.dot((a + x) * scale, W[...], preferred_element_type=jnp.float32)
    y = jnp.maximum(y, 0.0)
    if nout == 1:
        outs[0][...] = y
    else:
        w = y.shape[1] // nout
        for q in range(nout):
            outs[q][...] = y[:, q * w:(q + 1) * w]


def _make_tc_layer(nchunk, nout, bn=1000):
    grid = (NN // bn,)
    in_specs = [
        pl.BlockSpec((nchunk, bn, CHUNK), lambda i: (0, i, 0)),
        pl.BlockSpec((NC, bn, CHUNK), lambda i: (0, i, 0)),
    ]
    in_specs += [pl.BlockSpec((bn, CHUNK), lambda i: (i, 0))
                 for _ in range(nchunk)]
    in_specs += [
        pl.BlockSpec((nchunk * CHUNK, D_H), lambda i: (0, 0)),
    ]
    if nout == 1:
        out_shape = jax.ShapeDtypeStruct((NN, D_H), jnp.float32)
        out_specs = pl.BlockSpec((bn, D_H), lambda i: (i, 0))
    else:
        w = D_H // nout
        out_shape = [jax.ShapeDtypeStruct((NN, w), jnp.float32)
                     for _ in range(nout)]
        out_specs = [pl.BlockSpec((bn, w), lambda i: (i, 0))
                     for _ in range(nout)]

    def body(*refs):
        _tc_layer_body(nchunk, nout, refs)

    return pl.pallas_call(
        body, grid=grid, in_specs=in_specs,
        out_specs=out_specs, out_shape=out_shape)


@jax.jit
def kernel(in_feat, edge_index, edge_weights, W1, W2):
    src = edge_index[0]
    dst = edge_index[1]
    ew = edge_weights

    x0 = in_feat[:, :CHUNK]
    x1 = in_feat[:, CHUNK:]

    sc1 = _make_sc_aggregate(2, True)
    out1 = sc1(x0, x1, src, dst, ew)
    agg1 = out1[:2]
    degp = out1[2:]

    tc1 = _make_tc_layer(2, 4)
    h1c = tc1(agg1, degp, x0, x1, W1)

    sc2 = _make_sc_aggregate(4, False)
    agg2 = sc2(h1c[0], h1c[1], h1c[2], h1c[3], src, dst, ew)

    tc2 = _make_tc_layer(4, 1)
    out = tc2(agg2, degp, h1c[0], h1c[1], h1c[2], h1c[3], W2)
    return out
